# baseline (speedup 1.0000x reference)
import numpy as np
import jax
import jax.numpy as jnp
from jax.experimental import pallas as pl
from jax.experimental.pallas import tpu as pltpu

B = 1024
ATOMS = 64
REST = 64
NB = 8
TD = 8
HE = 16
OF = 4
CUTOFF = 5.0
EPS_PER = (ATOMS * ATOMS - ATOMS) // 2
PAIRS = ATOMS * ATOMS
BR = 64
G = BR // 2

_IU, _JU = np.triu_indices(ATOMS, k=1)
_PIDX_U = np.asarray(_IU * ATOMS + _JU, dtype=np.int32)
_PIDX_L = np.asarray(_JU * ATOMS + _IU, dtype=np.int32)


def _silu(v):
    return v * jax.nn.sigmoid(v)


def _rne(v):
    u = jax.lax.bitcast_convert_type(v, jnp.uint32)
    r = (u + jnp.uint32(0x7FFF) + ((u >> 16) & jnp.uint32(1))) & jnp.uint32(0xFFFF0000)
    return jax.lax.bitcast_convert_type(r, jnp.float32)


def _r(v):
    return v.astype(jnp.bfloat16).astype(jnp.float32)


def _mm(a, b):
    return jax.lax.dot_general(a.astype(jnp.bfloat16), b,
                               (((1,), (0,)), ((), ())),
                               preferred_element_type=jnp.float32)


def _pair_d2(px, py, pz):
    dx = px[:, None, :] - px[:, :, None]
    dy = py[:, None, :] - py[:, :, None]
    dz = pz[:, None, :] - pz[:, :, None]
    return dx * dx + dy * dy + dz * dz


def _body(px_ref, py_ref, pz_ref, xr_ref, esM_ref, esMT_ref, edM_ref, edMT_ref,
          We1_ref, be1_ref, Wn_ref, bn_ref, We2_ref, be2_ref,
          Wd1h_ref, Wd1dU_ref, Wd1dL_ref, bd1_ref, Wd2_ref, bd2_ref,
          Wd3_ref, bd3_ref, out_ref):
    px = px_ref[...]
    py = py_ref[...]
    pz = pz_ref[...]
    ii = jax.lax.broadcasted_iota(jnp.int32, (1, ATOMS, ATOMS), 1)
    jj = jax.lax.broadcasted_iota(jnp.int32, (1, ATOMS, ATOMS), 2)
    upper = ii < jj
    lower = ii > jj

    d2U = _pair_d2(px[:G], py[:G], pz[:G])
    d2L = _pair_d2(px[G:], py[G:], pz[G:])
    d2 = jnp.where(upper, d2U, d2L)
    d = jnp.sqrt(d2 + 1e-12)
    u = jnp.clip(d * (1.0 / CUTOFF), 1e-4, 1.0)
    env_over_u = (1.0 - u) * (1.0 - u) * (1.0 + 2.0 * u) / u
    t = jnp.float32(np.pi) * u
    s1 = jnp.sin(t)
    c1 = jnp.cos(t)
    rbf = [_r(s1 * env_over_u)]
    s, c = s1, c1
    for _ in range(NB - 1):
        s, c = s * c1 + c * s1, c * c1 - s * s1
        rbf.append(_r(s * env_over_u))

    esM = esM_ref[...]
    esMT = esMT_ref[...]
    edM = edM_ref[...]
    edMT = edMT_ref[...]

    hs = []
    for ch in range(HE):
        hp = rbf[0] * We1_ref[0, ch]
        for n in range(1, NB):
            hp = hp + rbf[n] * We1_ref[n, ch]
        embU = esM[:, ch:ch + 1][None] + edMT[ch:ch + 1, :][None]
        embL = esMT[ch:ch + 1, :][None] + edM[:, ch:ch + 1][None]
        emb = jnp.where(upper, embU, embL) + be1_ref[ch]
        hs.append(_silu(hp + emb))

    nodesU = [_r(jnp.sum(jnp.where(upper, h, 0.0), axis=1)) for h in hs]
    nodesL = [_r(jnp.sum(jnp.where(lower, h, 0.0), axis=2)) for h in hs]

    nodes = [jnp.concatenate([nU, nL], axis=0) for nU, nL in zip(nodesU, nodesL)]
    gatesU, gatesL = [], []
    for ch in range(HE):
        g = nodes[0] * Wn_ref[0, ch]
        for c2 in range(1, HE):
            g = g + nodes[c2] * Wn_ref[c2, ch]
        g = _silu(g + bn_ref[ch])
        gatesU.append(g[:G])
        gatesL.append(g[G:])

    hg = [_r(hs[ch] * jnp.where(upper, gatesU[ch][:, :, None],
                                gatesL[ch][:, None, :]))
          for ch in range(HE)]
    accU = _mm(xr_ref[:G], Wd1h_ref[...])
    accL = _mm(xr_ref[G:], Wd1h_ref[...])
    for f in range(OF):
        eo = hg[0] * We2_ref[0, f]
        for ch in range(1, HE):
            eo = eo + hg[ch] * We2_ref[ch, f]
        eo16 = (eo + be2_ref[f]).reshape(G, PAIRS).astype(jnp.bfloat16)
        accU = accU + jax.lax.dot_general(
            eo16, Wd1dU_ref[f], (((1,), (0,)), ((), ())),
            preferred_element_type=jnp.float32)
        accL = accL + jax.lax.dot_general(
            eo16, Wd1dL_ref[f], (((1,), (0,)), ((), ())),
            preferred_element_type=jnp.float32)

    acc = jnp.concatenate([accU, accL], axis=0)
    z = _silu(acc + bd1_ref[...])
    z = _silu(_mm(z, Wd2_ref[...]) + bd2_ref[...])
    out_ref[...] = _mm(z, Wd3_ref[...]) + bd3_ref[...]


def kernel(x, type_embed, We1, be1, Wn, bn, We2, be2, Wd1, bd1, Wd2, bd2, Wd3, bd3):
    f32, bf16 = jnp.float32, jnp.bfloat16
    xr = x[:, :REST]
    pos = x[:, REST:].reshape(B, ATOMS, 3)
    px, py, pz = pos[:, :, 0], pos[:, :, 1], pos[:, :, 2]
    type_embed = _rne(type_embed)
    We1 = _rne(We1)
    Wn = _rne(Wn)
    We2 = _rne(We2)
    esM = jax.lax.dot_general(type_embed.astype(bf16),
                              We1[NB:NB + TD].astype(bf16),
                              (((1,), (0,)), ((), ())),
                              preferred_element_type=f32)
    edM = jax.lax.dot_general(type_embed.astype(bf16),
                              We1[NB + TD:].astype(bf16),
                              (((1,), (0,)), ((), ())),
                              preferred_element_type=f32)
    Wd1h = Wd1[:REST].astype(bf16)
    wed = Wd1[REST:].reshape(EPS_PER, OF, 128).transpose(1, 0, 2)
    Wd1dU = jnp.zeros((OF, PAIRS, 128), f32).at[:, _PIDX_U, :].set(wed).astype(bf16)
    Wd1dL = jnp.zeros((OF, PAIRS, 128), f32).at[:, _PIDX_L, :].set(wed).astype(bf16)

    row_spec = pl.BlockSpec((BR, ATOMS), lambda i: (i, 0))
    const = lambda shape: pl.BlockSpec(shape, lambda i: (0,) * len(shape))
    smem = pl.BlockSpec(memory_space=pltpu.SMEM)

    out = pl.pallas_call(
        _body,
        grid=(B // BR,),
        in_specs=[
            row_spec, row_spec, row_spec, row_spec,
            const((ATOMS, HE)), const((HE, ATOMS)),
            const((ATOMS, HE)), const((HE, ATOMS)),
            smem, smem, smem, smem, smem, smem,
            const((REST, 128)),
            const((OF, PAIRS, 128)), const((OF, PAIRS, 128)),
            const((1, 128)), const((128, 128)), const((1, 128)),
            const((128, 64)), const((1, 64)),
        ],
        out_specs=pl.BlockSpec((BR, 64), lambda i: (i, 0)),
        out_shape=jax.ShapeDtypeStruct((B, 64), jnp.float32),
        compiler_params=pltpu.CompilerParams(
            dimension_semantics=("parallel",)),
    )(px, py, pz, xr, esM, esM.T, edM, edM.T, We1, be1, Wn, bn, We2, be2,
      Wd1h, Wd1dU, Wd1dL, bd1.reshape(1, 128), Wd2.astype(bf16),
      bd2.reshape(1, 128), Wd3.astype(bf16), bd3.reshape(1, 64))
    return out


# speedup vs baseline: 1.1406x; 1.1406x over previous
import numpy as np
import jax
import jax.numpy as jnp
from jax.experimental import pallas as pl
from jax.experimental.pallas import tpu as pltpu

B = 1024
ATOMS = 64
REST = 64
NB = 8
TD = 8
HE = 16
OF = 4
CUTOFF = 5.0
EPS_PER = (ATOMS * ATOMS - ATOMS) // 2
PAIRS = ATOMS * ATOMS
BR = 32
G = BR // 2

_IU, _JU = np.triu_indices(ATOMS, k=1)
_PIDX_U = np.asarray(_IU * ATOMS + _JU, dtype=np.int32)
_PIDX_L = np.asarray(_JU * ATOMS + _IU, dtype=np.int32)


def _silu(v):
    return v * jax.nn.sigmoid(v)


def _rne(v):
    u = jax.lax.bitcast_convert_type(v, jnp.uint32)
    r = (u + jnp.uint32(0x7FFF) + ((u >> 16) & jnp.uint32(1))) & jnp.uint32(0xFFFF0000)
    return jax.lax.bitcast_convert_type(r, jnp.float32)


def _r(v):
    return v.astype(jnp.bfloat16).astype(jnp.float32)


def _mm(a, b):
    return jax.lax.dot_general(a.astype(jnp.bfloat16), b,
                               (((1,), (0,)), ((), ())),
                               preferred_element_type=jnp.float32)


def _pair_d2(px, py, pz):
    dx = px[:, None, :] - px[:, :, None]
    dy = py[:, None, :] - py[:, :, None]
    dz = pz[:, None, :] - pz[:, :, None]
    return dx * dx + dy * dy + dz * dz


def _body(px_ref, py_ref, pz_ref, xr_ref, esM_ref, esMT_ref, edM_ref, edMT_ref,
          We1_ref, be1_ref, Wn_ref, bn_ref, We2_ref, be2_ref,
          Wd1h_ref, Wd1dU_ref, Wd1dL_ref, bd1_ref, Wd2_ref, bd2_ref,
          Wd3_ref, bd3_ref, out_ref):
    px = px_ref[...]
    py = py_ref[...]
    pz = pz_ref[...]
    ii = jax.lax.broadcasted_iota(jnp.int32, (1, ATOMS, ATOMS), 1)
    jj = jax.lax.broadcasted_iota(jnp.int32, (1, ATOMS, ATOMS), 2)
    upper = ii < jj
    lower = ii > jj

    d2U = _pair_d2(px[:G], py[:G], pz[:G])
    d2L = _pair_d2(px[G:], py[G:], pz[G:])
    d2 = jnp.where(upper, d2U, d2L)
    d = jnp.sqrt(d2 + 1e-12)
    u = jnp.clip(d * (1.0 / CUTOFF), 1e-4, 1.0)
    env_over_u = (1.0 - u) * (1.0 - u) * (1.0 + 2.0 * u) / u
    t = jnp.float32(np.pi) * u
    s1 = jnp.sin(t)
    c1 = jnp.cos(t)
    rbf = [_r(s1 * env_over_u)]
    s, c = s1, c1
    for _ in range(NB - 1):
        s, c = s * c1 + c * s1, c * c1 - s * s1
        rbf.append(_r(s * env_over_u))

    esM = esM_ref[...]
    esMT = esMT_ref[...]
    edM = edM_ref[...]
    edMT = edMT_ref[...]

    hs = []
    for ch in range(HE):
        hp = rbf[0] * We1_ref[0, ch]
        for n in range(1, NB):
            hp = hp + rbf[n] * We1_ref[n, ch]
        embU = esM[:, ch:ch + 1][None] + edMT[ch:ch + 1, :][None]
        embL = esMT[ch:ch + 1, :][None] + edM[:, ch:ch + 1][None]
        emb = jnp.where(upper, embU, embL) + be1_ref[ch]
        hs.append(_silu(hp + emb))

    nodesU = [_r(jnp.sum(jnp.where(upper, h, 0.0), axis=1)) for h in hs]
    nodesL = [_r(jnp.sum(jnp.where(lower, h, 0.0), axis=2)) for h in hs]

    nodes = [jnp.concatenate([nU, nL], axis=0) for nU, nL in zip(nodesU, nodesL)]
    gatesU, gatesL = [], []
    for ch in range(HE):
        g = nodes[0] * Wn_ref[0, ch]
        for c2 in range(1, HE):
            g = g + nodes[c2] * Wn_ref[c2, ch]
        g = _silu(g + bn_ref[ch])
        gatesU.append(g[:G])
        gatesL.append(g[G:])

    hg = [_r(hs[ch] * jnp.where(upper, gatesU[ch][:, :, None],
                                gatesL[ch][:, None, :]))
          for ch in range(HE)]
    accU = _mm(xr_ref[:G], Wd1h_ref[...])
    accL = _mm(xr_ref[G:], Wd1h_ref[...])
    for f in range(OF):
        eo = hg[0] * We2_ref[0, f]
        for ch in range(1, HE):
            eo = eo + hg[ch] * We2_ref[ch, f]
        eo16 = (eo + be2_ref[f]).reshape(G, PAIRS).astype(jnp.bfloat16)
        accU = accU + jax.lax.dot_general(
            eo16, Wd1dU_ref[f], (((1,), (0,)), ((), ())),
            preferred_element_type=jnp.float32)
        accL = accL + jax.lax.dot_general(
            eo16, Wd1dL_ref[f], (((1,), (0,)), ((), ())),
            preferred_element_type=jnp.float32)

    acc = jnp.concatenate([accU, accL], axis=0)
    z = _silu(acc + bd1_ref[...])
    z = _silu(_mm(z, Wd2_ref[...]) + bd2_ref[...])
    out_ref[...] = _mm(z, Wd3_ref[...]) + bd3_ref[...]


def kernel(x, type_embed, We1, be1, Wn, bn, We2, be2, Wd1, bd1, Wd2, bd2, Wd3, bd3):
    f32, bf16 = jnp.float32, jnp.bfloat16
    xr = x[:, :REST]
    pos = x[:, REST:].reshape(B, ATOMS, 3)
    px, py, pz = pos[:, :, 0], pos[:, :, 1], pos[:, :, 2]
    type_embed = _rne(type_embed)
    We1 = _rne(We1)
    Wn = _rne(Wn)
    We2 = _rne(We2)
    esM = jax.lax.dot_general(type_embed.astype(bf16),
                              We1[NB:NB + TD].astype(bf16),
                              (((1,), (0,)), ((), ())),
                              preferred_element_type=f32)
    edM = jax.lax.dot_general(type_embed.astype(bf16),
                              We1[NB + TD:].astype(bf16),
                              (((1,), (0,)), ((), ())),
                              preferred_element_type=f32)
    Wd1h = Wd1[:REST].astype(bf16)
    wed = Wd1[REST:].reshape(EPS_PER, OF, 128).transpose(1, 0, 2)
    Wd1dU = jnp.zeros((OF, PAIRS, 128), f32).at[:, _PIDX_U, :].set(wed).astype(bf16)
    Wd1dL = jnp.zeros((OF, PAIRS, 128), f32).at[:, _PIDX_L, :].set(wed).astype(bf16)

    row_spec = pl.BlockSpec((BR, ATOMS), lambda i: (i, 0))
    const = lambda shape: pl.BlockSpec(shape, lambda i: (0,) * len(shape))
    smem = pl.BlockSpec(memory_space=pltpu.SMEM)

    out = pl.pallas_call(
        _body,
        grid=(B // BR,),
        in_specs=[
            row_spec, row_spec, row_spec, row_spec,
            const((ATOMS, HE)), const((HE, ATOMS)),
            const((ATOMS, HE)), const((HE, ATOMS)),
            smem, smem, smem, smem, smem, smem,
            const((REST, 128)),
            const((OF, PAIRS, 128)), const((OF, PAIRS, 128)),
            const((1, 128)), const((128, 128)), const((1, 128)),
            const((128, 64)), const((1, 64)),
        ],
        out_specs=pl.BlockSpec((BR, 64), lambda i: (i, 0)),
        out_shape=jax.ShapeDtypeStruct((B, 64), jnp.float32),
        compiler_params=pltpu.CompilerParams(
            dimension_semantics=("parallel",)),
    )(px, py, pz, xr, esM, esM.T, edM, edM.T, We1, be1, Wn, bn, We2, be2,
      Wd1h, Wd1dU, Wd1dL, bd1.reshape(1, 128), Wd2.astype(bf16),
      bd2.reshape(1, 128), Wd3.astype(bf16), bd3.reshape(1, 64))
    return out
